# Initial kernel scaffold; baseline (speedup 1.0000x reference)
import jax
import jax.numpy as jnp
from jax.experimental import pallas as pl


def kernel(feature_list, adj_list, aug_pun_adj, pooled_output, p_nodes_mask, o_nodes_mask, W_rel0, W_root0, bias0, W_rel1, W_root1, bias1):
    raise NotImplementedError("write your pallas kernel here")


# speedup vs baseline: 663.9857x; 663.9857x over previous
import functools

import jax
import jax.numpy as jnp
from jax.experimental import pallas as pl

_BS, _NN, _D = 2, 512, 128
_NUM_REL = 2


def _rgcn_kernel(adj_ref, x_ref, wrel0_ref, wroot0_ref, b0_ref,
                 wrel1_ref, wroot1_ref, b1_ref, out_ref):
    aug = adj_ref[0, 0]
    pun = adj_ref[1, 0]
    a1 = (aug == 1).astype(jnp.float32)
    a0 = ((pun == 1) & (aug != 1)).astype(jnp.float32)

    inv0 = 1.0 / jnp.maximum(jnp.sum(a0, axis=0), 1.0)
    inv1 = 1.0 / jnp.maximum(jnp.sum(a1, axis=0), 1.0)

    x = x_ref[0]
    hi = jax.lax.Precision.HIGHEST
    contract = (((0,), (0,)), ((), ()))

    for wrel_ref, wroot_ref, b_ref in ((wrel0_ref, wroot0_ref, b0_ref),
                                       (wrel1_ref, wroot1_ref, b1_ref)):
        h = jnp.dot(x, wroot_ref[...], precision=hi,
                    preferred_element_type=jnp.float32) + b_ref[...]
        agg0 = jax.lax.dot_general(a0, x, contract, precision=hi,
                                   preferred_element_type=jnp.float32)
        agg1 = jax.lax.dot_general(a1, x, contract, precision=hi,
                                   preferred_element_type=jnp.float32)
        h = h + jnp.dot(agg0 * inv0[:, None], wrel_ref[0], precision=hi,
                        preferred_element_type=jnp.float32)
        h = h + jnp.dot(agg1 * inv1[:, None], wrel_ref[1], precision=hi,
                        preferred_element_type=jnp.float32)
        x = jnp.where(h > 0, h, jnp.exp(jnp.minimum(h, 0.0)) - 1.0)

    out_ref[0] = x


@functools.partial(jax.jit, static_argnames=())
def _run(adj, x, wrel0, wroot0, b0, wrel1, wroot1, b1):
    grid = (_BS,)
    return pl.pallas_call(
        _rgcn_kernel,
        grid=grid,
        in_specs=[
            pl.BlockSpec((2, 1, _NN, _NN), lambda b: (0, b, 0, 0)),
            pl.BlockSpec((1, _NN, _D), lambda b: (b, 0, 0)),
            pl.BlockSpec((_NUM_REL, _D, _D), lambda b: (0, 0, 0)),
            pl.BlockSpec((_D, _D), lambda b: (0, 0)),
            pl.BlockSpec((1, _D), lambda b: (0, 0)),
            pl.BlockSpec((_NUM_REL, _D, _D), lambda b: (0, 0, 0)),
            pl.BlockSpec((_D, _D), lambda b: (0, 0)),
            pl.BlockSpec((1, _D), lambda b: (0, 0)),
        ],
        out_specs=pl.BlockSpec((1, _NN, _D), lambda b: (b, 0, 0)),
        out_shape=jax.ShapeDtypeStruct((_BS, _NN, _D), jnp.float32),
    )(adj, x, wrel0, wroot0, b0, wrel1, wroot1, b1)


def kernel(feature_list, adj_list, aug_pun_adj, pooled_output, p_nodes_mask,
           o_nodes_mask, W_rel0, W_root0, bias0, W_rel1, W_root1, bias1):
    x = feature_list[0]
    adj = aug_pun_adj.astype(jnp.int32)
    out = _run(adj, x, W_rel0, W_root0, bias0.reshape(1, _D),
               W_rel1, W_root1, bias1.reshape(1, _D))
    return out


# speedup vs baseline: 1096.3648x; 1.6512x over previous
import functools

import jax
import jax.numpy as jnp
from jax.experimental import pallas as pl

_BS, _NN, _D = 2, 512, 128
_NUM_REL = 2


def _rgcn_kernel(adj_ref, x_ref, wrel0_ref, wroot0_ref, b0_ref,
                 wrel1_ref, wroot1_ref, b1_ref, out_ref):
    aug = adj_ref[0, 0]
    pun = adj_ref[1, 0]
    m1 = aug == 1
    m0 = (pun == 1) & (aug != 1)
    a1 = m1.astype(jnp.bfloat16)
    a0 = m0.astype(jnp.bfloat16)

    inv0 = 1.0 / jnp.maximum(jnp.sum(m0.astype(jnp.float32), axis=0), 1.0)
    inv1 = 1.0 / jnp.maximum(jnp.sum(m1.astype(jnp.float32), axis=0), 1.0)

    x = x_ref[0]
    contract = (((0,), (0,)), ((), ()))

    def split(v):
        vh = v.astype(jnp.bfloat16)
        vl = (v - vh.astype(jnp.float32)).astype(jnp.bfloat16)
        return vh, vl

    def agg(a, xh, xl):
        s = jax.lax.dot_general(a, xh, contract,
                                preferred_element_type=jnp.float32)
        return s + jax.lax.dot_general(a, xl, contract,
                                       preferred_element_type=jnp.float32)

    def mm3(u, wh, wl):
        uh, ul = split(u)
        return (jnp.dot(uh, wh, preferred_element_type=jnp.float32)
                + jnp.dot(uh, wl, preferred_element_type=jnp.float32)
                + jnp.dot(ul, wh, preferred_element_type=jnp.float32))

    for wrel_ref, wroot_ref, b_ref in ((wrel0_ref, wroot0_ref, b0_ref),
                                       (wrel1_ref, wroot1_ref, b1_ref)):
        wrh, wrl = split(wroot_ref[...])
        w0h, w0l = split(wrel_ref[0])
        w1h, w1l = split(wrel_ref[1])
        xh, xl = split(x)
        h = mm3(x, wrh, wrl) + b_ref[...]
        h = h + mm3(agg(a0, xh, xl) * inv0[:, None], w0h, w0l)
        h = h + mm3(agg(a1, xh, xl) * inv1[:, None], w1h, w1l)
        x = jnp.where(h > 0, h, jnp.exp(jnp.minimum(h, 0.0)) - 1.0)

    out_ref[0] = x


@functools.partial(jax.jit, static_argnames=())
def _run(adj, x, wrel0, wroot0, b0, wrel1, wroot1, b1):
    grid = (_BS,)
    return pl.pallas_call(
        _rgcn_kernel,
        grid=grid,
        in_specs=[
            pl.BlockSpec((2, 1, _NN, _NN), lambda b: (0, b, 0, 0)),
            pl.BlockSpec((1, _NN, _D), lambda b: (b, 0, 0)),
            pl.BlockSpec((_NUM_REL, _D, _D), lambda b: (0, 0, 0)),
            pl.BlockSpec((_D, _D), lambda b: (0, 0)),
            pl.BlockSpec((1, _D), lambda b: (0, 0)),
            pl.BlockSpec((_NUM_REL, _D, _D), lambda b: (0, 0, 0)),
            pl.BlockSpec((_D, _D), lambda b: (0, 0)),
            pl.BlockSpec((1, _D), lambda b: (0, 0)),
        ],
        out_specs=pl.BlockSpec((1, _NN, _D), lambda b: (b, 0, 0)),
        out_shape=jax.ShapeDtypeStruct((_BS, _NN, _D), jnp.float32),
    )(adj, x, wrel0, wroot0, b0, wrel1, wroot1, b1)


def kernel(feature_list, adj_list, aug_pun_adj, pooled_output, p_nodes_mask,
           o_nodes_mask, W_rel0, W_root0, bias0, W_rel1, W_root1, bias1):
    x = feature_list[0]
    adj = aug_pun_adj.astype(jnp.int32)
    out = _run(adj, x, W_rel0, W_root0, bias0.reshape(1, _D),
               W_rel1, W_root1, bias1.reshape(1, _D))
    return out
